# Initial kernel scaffold; baseline (speedup 1.0000x reference)
import functools

import jax
import jax.numpy as jnp
from jax import lax
from jax.experimental import pallas as pl
from jax.experimental.pallas import tpu as pltpu
from jax.experimental.pallas import tpu_sc as plsc

NUM_SOURCES = 100000
D = 128
NUM_NEURONS = 32768
NUM_CONN = 524288

NC = 2
NS = 16
K = 128
NB = NUM_CONN // K
NQ = 4
SEG_Q = NUM_NEURONS // NQ
ROWS_PER_SUB = SEG_Q // NS
TRASH = SEG_Q
ACC_ROWS = SEG_Q + 8


def _sc_gather_segment_sum(prev_values, gather_idx, segment_ids, bounds, zeros):
    mesh = plsc.VectorSubcoreMesh(core_axis_name="c", subcore_axis_name="s")

    @functools.partial(
        pl.kernel,
        mesh=mesh,
        out_type=jax.ShapeDtypeStruct((NUM_NEURONS, D), jnp.float32),
        scratch_types=[
            pltpu.VMEM((K,), jnp.int32),
            pltpu.VMEM((K,), jnp.int32),
            pltpu.VMEM((K,), jnp.int32),
            pltpu.VMEM((K, D), jnp.float32),
            pltpu.VMEM((16,), jnp.int32),
            pltpu.VMEM_SHARED((ACC_ROWS, D), jnp.float32),
            pltpu.SemaphoreType.DMA,
        ],
    )
    def k(prev_hbm, gidx_hbm, seg_hbm, bnd_hbm, zeros_hbm, out_hbm,
          gidx_v, seg_v, loc_v, rows_v, bnd_v, acc_sh, sem):
        cid = lax.axis_index("c")
        sid = lax.axis_index("s")
        pltpu.sync_copy(bnd_hbm, bnd_v)
        bv = bnd_v[...]
        lanes = lax.iota(jnp.int32, 16)

        for phase in range(NQ // NC):
            q = cid * (NQ // NC) + phase
            seg_base = q * SEG_Q
            b0 = jnp.sum(jnp.where(lanes == 2 * q, bv, 0))
            b1 = jnp.sum(jnp.where(lanes == 2 * q + 1, bv, 0))

            pltpu.sync_copy(zeros_hbm, acc_sh.at[pl.ds(sid * ROWS_PER_SUB, ROWS_PER_SUB)])
            plsc.subcore_barrier()

            nt = jnp.maximum(b1 - b0 - sid + NS - 1, 0) // NS

            def body(t, carry):
                off = (b0 + sid + t * NS) * K
                pltpu.sync_copy(gidx_hbm.at[pl.ds(off, K)], gidx_v)
                pltpu.sync_copy(seg_hbm.at[pl.ds(off, K)], seg_v)
                for j in range(K // 16):
                    sl = pl.ds(j * 16, 16)
                    lv = seg_v[sl] - seg_base
                    lv = jnp.where((lv < 0) | (lv >= SEG_Q), TRASH, lv)
                    loc_v[sl] = lv
                pltpu.async_copy(prev_hbm.at[gidx_v], rows_v, sem).wait()
                pltpu.sync_copy(rows_v, acc_sh.at[loc_v], add=True)
                return carry

            lax.fori_loop(0, nt, body, 0)
            plsc.subcore_barrier()

            pltpu.sync_copy(
                acc_sh.at[pl.ds(sid * ROWS_PER_SUB, ROWS_PER_SUB)],
                out_hbm.at[pl.ds(seg_base + sid * ROWS_PER_SUB, ROWS_PER_SUB)],
            )
            plsc.subcore_barrier()

    return k(prev_values, gather_idx, segment_ids, bounds, zeros)


def _tc_matmul_tanh(seg_sum, W):
    BM = 2048

    def body(s_ref, w_ref, o_ref):
        o_ref[...] = jnp.tanh(
            jnp.dot(s_ref[...], w_ref[...], preferred_element_type=jnp.float32)
        )

    return pl.pallas_call(
        body,
        grid=(NUM_NEURONS // BM,),
        in_specs=[
            pl.BlockSpec((BM, D), lambda i: (i, 0)),
            pl.BlockSpec((D, D), lambda i: (0, 0)),
        ],
        out_specs=pl.BlockSpec((BM, D), lambda i: (i, 0)),
        out_shape=jax.ShapeDtypeStruct((NUM_NEURONS, D), jnp.float32),
    )(seg_sum, W)


def kernel(prev_values, W, gather_idx, segment_ids):
    gidx = gather_idx.astype(jnp.int32)
    seg = segment_ids.astype(jnp.int32)

    edges = jnp.arange(1, NQ, dtype=jnp.int32) * SEG_Q
    cut = jnp.searchsorted(seg, edges, side="left").astype(jnp.int32)
    starts = jnp.concatenate([jnp.zeros((1,), jnp.int32), cut // K])
    ends = jnp.concatenate([(cut + K - 1) // K, jnp.full((1,), NB, jnp.int32)])
    bounds = jnp.concatenate(
        [jnp.stack([starts, ends], axis=1).reshape(-1),
         jnp.zeros((16 - 2 * NQ,), jnp.int32)]
    )
    zeros = jnp.zeros((ROWS_PER_SUB, D), jnp.float32)

    seg_sum = _sc_gather_segment_sum(prev_values, gidx, seg, bounds, zeros)
    return _tc_matmul_tanh(seg_sum, W)


# speedup vs baseline: 7.9002x; 7.9002x over previous
import functools

import jax
import jax.numpy as jnp
from jax import lax
from jax.experimental import pallas as pl
from jax.experimental.pallas import tpu as pltpu
from jax.experimental.pallas import tpu_sc as plsc

NUM_SOURCES = 100000
D = 128
NUM_NEURONS = 32768
NUM_CONN = 524288

NC = 2
NS = 16
K = 128
NB = NUM_CONN // K
NQ = 4
SEG_Q = NUM_NEURONS // NQ
ROWS_PER_SUB = SEG_Q // NS
TRASH = SEG_Q
ACC_ROWS = SEG_Q + 8


def _sc_gather_segment_sum(prev_values, gather_idx, segment_ids, bounds, zeros):
    mesh = plsc.VectorSubcoreMesh(core_axis_name="c", subcore_axis_name="s")

    @functools.partial(
        pl.kernel,
        mesh=mesh,
        out_type=jax.ShapeDtypeStruct((NUM_NEURONS, D), jnp.float32),
        scratch_types=[
            pltpu.VMEM((K,), jnp.int32),
            pltpu.VMEM((K,), jnp.int32),
            pltpu.VMEM((K,), jnp.int32),
            pltpu.VMEM((K, D), jnp.float32),
            pltpu.VMEM((16,), jnp.int32),
            pltpu.VMEM_SHARED((ACC_ROWS, D), jnp.float32),
            pltpu.SemaphoreType.DMA,
        ],
    )
    def k(prev_hbm, gidx_hbm, seg_hbm, bnd_hbm, zeros_hbm, out_hbm,
          gidx_v, seg_v, loc_v, rows_v, bnd_v, acc_sh, sem):
        cid = lax.axis_index("c")
        sid = lax.axis_index("s")
        pltpu.sync_copy(bnd_hbm.at[cid], bnd_v)
        bv = bnd_v[...]

        for phase in range(NQ // NC):
            q = cid * (NQ // NC) + phase
            seg_base = q * SEG_Q
            b0 = bv[2 * phase]
            b1 = bv[2 * phase + 1]

            pltpu.sync_copy(zeros_hbm, acc_sh.at[pl.ds(sid * ROWS_PER_SUB, ROWS_PER_SUB)])
            plsc.subcore_barrier()

            nt = jnp.maximum(b1 - b0 - sid + NS - 1, 0) // NS

            def body(t, carry):
                off = (b0 + sid + t * NS) * K
                pltpu.sync_copy(gidx_hbm.at[pl.ds(off, K)], gidx_v)
                pltpu.sync_copy(seg_hbm.at[pl.ds(off, K)], seg_v)
                for j in range(K // 16):
                    sl = pl.ds(j * 16, 16)
                    lv = seg_v[sl] - seg_base
                    lv = jnp.where((lv < 0) | (lv >= SEG_Q), TRASH, lv)
                    loc_v[sl] = lv
                pltpu.async_copy(prev_hbm.at[gidx_v], rows_v, sem).wait()
                pltpu.sync_copy(rows_v, acc_sh.at[loc_v], add=True)
                return carry

            lax.fori_loop(0, nt, body, 0)
            plsc.subcore_barrier()

            pltpu.sync_copy(
                acc_sh.at[pl.ds(sid * ROWS_PER_SUB, ROWS_PER_SUB)],
                out_hbm.at[pl.ds(seg_base + sid * ROWS_PER_SUB, ROWS_PER_SUB)],
            )
            plsc.subcore_barrier()

    return k(prev_values, gather_idx, segment_ids, bounds, zeros)


def _tc_matmul_tanh(seg_sum, W):
    BM = 2048

    def body(s_ref, w_ref, o_ref):
        o_ref[...] = jnp.tanh(
            jnp.dot(s_ref[...], w_ref[...], preferred_element_type=jnp.float32)
        )

    return pl.pallas_call(
        body,
        grid=(NUM_NEURONS // BM,),
        in_specs=[
            pl.BlockSpec((BM, D), lambda i: (i, 0)),
            pl.BlockSpec((D, D), lambda i: (0, 0)),
        ],
        out_specs=pl.BlockSpec((BM, D), lambda i: (i, 0)),
        out_shape=jax.ShapeDtypeStruct((NUM_NEURONS, D), jnp.float32),
    )(seg_sum, W)


def kernel(prev_values, W, gather_idx, segment_ids):
    gidx = gather_idx.astype(jnp.int32)
    seg = segment_ids.astype(jnp.int32)

    edges = jnp.arange(1, NQ, dtype=jnp.int32) * SEG_Q
    cut = jnp.searchsorted(seg, edges, side="left").astype(jnp.int32)
    starts = jnp.concatenate([jnp.zeros((1,), jnp.int32), cut // K])
    ends = jnp.concatenate([(cut + K - 1) // K, jnp.full((1,), NB, jnp.int32)])
    per_q = jnp.stack([starts, ends], axis=1).reshape(NC, 2 * (NQ // NC))
    bounds = jnp.concatenate(
        [per_q, jnp.zeros((NC, 16 - 2 * (NQ // NC)), jnp.int32)], axis=1
    )
    zeros = jnp.zeros((ROWS_PER_SUB, D), jnp.float32)

    seg_sum = _sc_gather_segment_sum(prev_values, gidx, seg, bounds, zeros)
    return _tc_matmul_tanh(seg_sum, W)


# speedup vs baseline: 11.8578x; 1.5010x over previous
import functools

import jax
import jax.numpy as jnp
from jax import lax
from jax.experimental import pallas as pl
from jax.experimental.pallas import tpu as pltpu
from jax.experimental.pallas import tpu_sc as plsc

NUM_SOURCES = 100000
D = 128
NUM_NEURONS = 32768
NUM_CONN = 524288

NC = 2
NS = 16
K = 128
NB = NUM_CONN // K
NQ = 4
SEG_Q = NUM_NEURONS // NQ
ROWS_PER_SUB = SEG_Q // NS
TRASH = SEG_Q
ACC_ROWS = SEG_Q + 8


def _sc_gather_segment_sum(prev_values, gather_idx, segment_ids, bounds, zeros):
    mesh = plsc.VectorSubcoreMesh(core_axis_name="c", subcore_axis_name="s")

    @functools.partial(
        pl.kernel,
        mesh=mesh,
        out_type=jax.ShapeDtypeStruct((NUM_NEURONS, D), jnp.float32),
        scratch_types=[
            pltpu.VMEM((2, K), jnp.int32),
            pltpu.VMEM((K,), jnp.int32),
            pltpu.VMEM((2, K), jnp.int32),
            pltpu.VMEM((2, K, D), jnp.float32),
            pltpu.VMEM((16,), jnp.int32),
            pltpu.VMEM_SHARED((ACC_ROWS, D), jnp.float32),
            pltpu.SemaphoreType.DMA,
            pltpu.SemaphoreType.DMA,
        ],
    )
    def k(prev_hbm, gidx_hbm, seg_hbm, bnd_hbm, zeros_hbm, out_hbm,
          gidx_v, seg_v, loc_v, rows_v, bnd_v, acc_sh, sem0, sem1):
        cid = lax.axis_index("c")
        sid = lax.axis_index("s")
        pltpu.sync_copy(bnd_hbm.at[cid], bnd_v)
        bv = bnd_v[...]
        sems = (sem0, sem1)

        for phase in range(NQ // NC):
            q = cid * (NQ // NC) + phase
            seg_base = q * SEG_Q
            b0 = bv[2 * phase]
            b1 = bv[2 * phase + 1]

            pltpu.sync_copy(zeros_hbm, acc_sh.at[pl.ds(sid * ROWS_PER_SUB, ROWS_PER_SUB)])
            plsc.subcore_barrier()

            nb = b1 - b0
            per = (nb + NS - 1) // NS
            s0 = b0 + sid * per
            cnt = jnp.clip(b1 - s0, 0, per)

            def launch(t, slot):
                off = (s0 + t) * K
                pltpu.sync_copy(gidx_hbm.at[pl.ds(off, K)], gidx_v.at[slot])
                pltpu.sync_copy(seg_hbm.at[pl.ds(off, K)], seg_v)
                for j in range(K // 16):
                    sl = pl.ds(j * 16, 16)
                    lv = seg_v[sl] - seg_base
                    lv = jnp.where((lv < 0) | (lv >= SEG_Q), TRASH, lv)
                    loc_v[slot, sl] = lv
                pltpu.async_copy(prev_hbm.at[gidx_v.at[slot]], rows_v.at[slot],
                                 sems[slot])

            def flush(slot):
                pltpu.make_async_copy(prev_hbm.at[gidx_v.at[slot]],
                                      rows_v.at[slot], sems[slot]).wait()
                pltpu.sync_copy(rows_v.at[slot], acc_sh.at[loc_v.at[slot]],
                                add=True)

            @pl.when(cnt > 0)
            def _prologue():
                launch(0, 0)

            def pair_body(p, carry):
                t0 = 2 * p
                t1 = t0 + 1

                @pl.when(t1 < cnt)
                def _():
                    launch(t1, 1)

                @pl.when(t0 < cnt)
                def _():
                    flush(0)

                @pl.when(t1 + 1 < cnt)
                def _():
                    launch(t1 + 1, 0)

                @pl.when(t1 < cnt)
                def _():
                    flush(1)

                return carry

            lax.fori_loop(0, (cnt + 1) // 2, pair_body, 0)
            plsc.subcore_barrier()

            pltpu.sync_copy(
                acc_sh.at[pl.ds(sid * ROWS_PER_SUB, ROWS_PER_SUB)],
                out_hbm.at[pl.ds(seg_base + sid * ROWS_PER_SUB, ROWS_PER_SUB)],
            )
            plsc.subcore_barrier()

    return k(prev_values, gather_idx, segment_ids, bounds, zeros)


def _tc_matmul_tanh(seg_sum, W):
    BM = 2048

    def body(s_ref, w_ref, o_ref):
        o_ref[...] = jnp.tanh(
            jnp.dot(s_ref[...], w_ref[...], preferred_element_type=jnp.float32)
        )

    return pl.pallas_call(
        body,
        grid=(NUM_NEURONS // BM,),
        in_specs=[
            pl.BlockSpec((BM, D), lambda i: (i, 0)),
            pl.BlockSpec((D, D), lambda i: (0, 0)),
        ],
        out_specs=pl.BlockSpec((BM, D), lambda i: (i, 0)),
        out_shape=jax.ShapeDtypeStruct((NUM_NEURONS, D), jnp.float32),
    )(seg_sum, W)


def kernel(prev_values, W, gather_idx, segment_ids):
    gidx = gather_idx.astype(jnp.int32)
    seg = segment_ids.astype(jnp.int32)

    edges = jnp.arange(1, NQ, dtype=jnp.int32) * SEG_Q
    cut = jnp.searchsorted(seg, edges, side="left").astype(jnp.int32)
    starts = jnp.concatenate([jnp.zeros((1,), jnp.int32), cut // K])
    ends = jnp.concatenate([(cut + K - 1) // K, jnp.full((1,), NB, jnp.int32)])
    per_q = jnp.stack([starts, ends], axis=1).reshape(NC, 2 * (NQ // NC))
    bounds = jnp.concatenate(
        [per_q, jnp.zeros((NC, 16 - 2 * (NQ // NC)), jnp.int32)], axis=1
    )
    zeros = jnp.zeros((ROWS_PER_SUB, D), jnp.float32)

    seg_sum = _sc_gather_segment_sum(prev_values, gidx, seg, bounds, zeros)
    return _tc_matmul_tanh(seg_sum, W)


# speedup vs baseline: 15.0401x; 1.2684x over previous
import functools

import jax
import jax.numpy as jnp
from jax import lax
from jax.experimental import pallas as pl
from jax.experimental.pallas import tpu as pltpu
from jax.experimental.pallas import tpu_sc as plsc

NUM_SOURCES = 100000
D = 128
NUM_NEURONS = 32768
NUM_CONN = 524288

NC = 2
NS = 16
K = 128
NB = NUM_CONN // K
NQ = 4
SEG_Q = NUM_NEURONS // NQ
ROWS_PER_SUB = SEG_Q // NS
TRASH = SEG_Q
ACC_ROWS = SEG_Q + 8
R = 3


def _sc_gather_segment_sum(prev_values, gather_idx, segment_ids, bounds, zeros):
    mesh = plsc.VectorSubcoreMesh(core_axis_name="c", subcore_axis_name="s")

    @functools.partial(
        pl.kernel,
        mesh=mesh,
        out_type=jax.ShapeDtypeStruct((NUM_NEURONS, D), jnp.float32),
        scratch_types=[
            pltpu.VMEM((R, K), jnp.int32),
            pltpu.VMEM((R, K), jnp.int32),
            pltpu.VMEM((R, K), jnp.int32),
            pltpu.VMEM((R, K, D), jnp.float32),
            pltpu.VMEM((16,), jnp.int32),
            pltpu.VMEM_SHARED((ACC_ROWS, D), jnp.float32),
            pltpu.SemaphoreType.DMA((R,)),
            pltpu.SemaphoreType.DMA((R,)),
            pltpu.SemaphoreType.DMA((R,)),
        ],
    )
    def k(prev_hbm, gidx_hbm, seg_hbm, bnd_hbm, zeros_hbm, out_hbm,
          gidx_v, seg_v, loc_v, rows_v, bnd_v, acc_sh, isem, gsem, ssem):
        cid = lax.axis_index("c")
        sid = lax.axis_index("s")
        pltpu.sync_copy(bnd_hbm.at[cid], bnd_v)
        bv = bnd_v[...]

        for phase in range(NQ // NC):
            q = cid * (NQ // NC) + phase
            seg_base = q * SEG_Q
            b0 = bv[2 * phase]
            b1 = bv[2 * phase + 1]

            pltpu.sync_copy(zeros_hbm, acc_sh.at[pl.ds(sid * ROWS_PER_SUB, ROWS_PER_SUB)])
            plsc.subcore_barrier()

            nb = b1 - b0
            per = (nb + NS - 1) // NS
            s0 = b0 + sid * per
            cnt = jnp.clip(b1 - s0, 0, per)

            def stage_a(t, slot):
                off = (s0 + t) * K
                pltpu.async_copy(gidx_hbm.at[pl.ds(off, K)], gidx_v.at[slot],
                                 isem.at[slot])
                pltpu.async_copy(seg_hbm.at[pl.ds(off, K)], seg_v.at[slot],
                                 isem.at[slot])

            def stage_b(t, slot):
                off = (s0 + t) * K
                pltpu.make_async_copy(gidx_hbm.at[pl.ds(off, K)],
                                      gidx_v.at[slot], isem.at[slot]).wait()
                pltpu.make_async_copy(seg_hbm.at[pl.ds(off, K)],
                                      seg_v.at[slot], isem.at[slot]).wait()

                @pl.when(t >= R)
                def _():
                    pltpu.make_async_copy(rows_v.at[slot],
                                          acc_sh.at[loc_v.at[slot]],
                                          ssem.at[slot]).wait()

                for j in range(K // 16):
                    sl = pl.ds(j * 16, 16)
                    lv = seg_v[slot, sl] - seg_base
                    lv = jnp.where((lv < 0) | (lv >= SEG_Q), TRASH, lv)
                    loc_v[slot, sl] = lv
                pltpu.async_copy(prev_hbm.at[gidx_v.at[slot]], rows_v.at[slot],
                                 gsem.at[slot])

            def stage_c(t, slot):
                pltpu.make_async_copy(prev_hbm.at[gidx_v.at[slot]],
                                      rows_v.at[slot], gsem.at[slot]).wait()
                pltpu.async_copy(rows_v.at[slot], acc_sh.at[loc_v.at[slot]],
                                 ssem.at[slot], add=True)

            def block_body(p, carry):
                for r in range(R):
                    t = p * R + r

                    @pl.when(t + 2 < cnt)
                    def _():
                        stage_a(t + 2, (r + 2) % R)

                    @pl.when(t + 1 < cnt)
                    def _():
                        stage_b(t + 1, (r + 1) % R)

                    @pl.when(t < cnt)
                    def _():
                        stage_c(t, r)

                return carry

            @pl.when(cnt > 0)
            def _():
                stage_a(0, 0)

            @pl.when(cnt > 1)
            def _():
                stage_a(1, 1)

            @pl.when(cnt > 0)
            def _():
                stage_b(0, 0)

            lax.fori_loop(0, (cnt + R - 1) // R, block_body, 0)

            for r in range(R):
                @pl.when(r < cnt)
                def _():
                    pltpu.make_async_copy(rows_v.at[r],
                                          acc_sh.at[loc_v.at[r]],
                                          ssem.at[r]).wait()

            plsc.subcore_barrier()

            pltpu.sync_copy(
                acc_sh.at[pl.ds(sid * ROWS_PER_SUB, ROWS_PER_SUB)],
                out_hbm.at[pl.ds(seg_base + sid * ROWS_PER_SUB, ROWS_PER_SUB)],
            )
            plsc.subcore_barrier()

    return k(prev_values, gather_idx, segment_ids, bounds, zeros)


def _tc_matmul_tanh(seg_sum, W):
    BM = 2048

    def body(s_ref, w_ref, o_ref):
        o_ref[...] = jnp.tanh(
            jnp.dot(s_ref[...], w_ref[...], preferred_element_type=jnp.float32)
        )

    return pl.pallas_call(
        body,
        grid=(NUM_NEURONS // BM,),
        in_specs=[
            pl.BlockSpec((BM, D), lambda i: (i, 0)),
            pl.BlockSpec((D, D), lambda i: (0, 0)),
        ],
        out_specs=pl.BlockSpec((BM, D), lambda i: (i, 0)),
        out_shape=jax.ShapeDtypeStruct((NUM_NEURONS, D), jnp.float32),
    )(seg_sum, W)


def kernel(prev_values, W, gather_idx, segment_ids):
    gidx = gather_idx.astype(jnp.int32)
    seg = segment_ids.astype(jnp.int32)

    edges = jnp.arange(1, NQ, dtype=jnp.int32) * SEG_Q
    cut = jnp.searchsorted(seg, edges, side="left").astype(jnp.int32)
    starts = jnp.concatenate([jnp.zeros((1,), jnp.int32), cut // K])
    ends = jnp.concatenate([(cut + K - 1) // K, jnp.full((1,), NB, jnp.int32)])
    per_q = jnp.stack([starts, ends], axis=1).reshape(NC, 2 * (NQ // NC))
    bounds = jnp.concatenate(
        [per_q, jnp.zeros((NC, 16 - 2 * (NQ // NC)), jnp.int32)], axis=1
    )
    zeros = jnp.zeros((ROWS_PER_SUB, D), jnp.float32)

    seg_sum = _sc_gather_segment_sum(prev_values, gidx, seg, bounds, zeros)
    return _tc_matmul_tanh(seg_sum, W)


# speedup vs baseline: 15.1123x; 1.0048x over previous
import functools

import jax
import jax.numpy as jnp
from jax import lax
from jax.experimental import pallas as pl
from jax.experimental.pallas import tpu as pltpu
from jax.experimental.pallas import tpu_sc as plsc

NUM_SOURCES = 100000
D = 128
NUM_NEURONS = 32768
NUM_CONN = 524288

NC = 2
NS = 16
K = 128
NB = NUM_CONN // K
NQ = 4
SEG_Q = NUM_NEURONS // NQ
ROWS_PER_SUB = SEG_Q // NS
TRASH = SEG_Q
ACC_ROWS = SEG_Q + 8
R = 3
GI = 6
GIK = GI * K
PAD = (GI - 1) * K


def _sc_gather_segment_sum(prev_values, gather_idx, segment_ids, bounds, zeros):
    mesh = plsc.VectorSubcoreMesh(core_axis_name="c", subcore_axis_name="s")

    @functools.partial(
        pl.kernel,
        mesh=mesh,
        out_type=jax.ShapeDtypeStruct((NUM_NEURONS, D), jnp.float32),
        scratch_types=[
            pltpu.VMEM((2, GIK), jnp.int32),
            pltpu.VMEM((2, GIK), jnp.int32),
            pltpu.VMEM((R, K), jnp.int32),
            pltpu.VMEM((R, K, D), jnp.float32),
            pltpu.VMEM((16,), jnp.int32),
            pltpu.VMEM_SHARED((ACC_ROWS, D), jnp.float32),
            pltpu.SemaphoreType.DMA((2,)),
            pltpu.SemaphoreType.DMA((R,)),
            pltpu.SemaphoreType.DMA((R,)),
        ],
    )
    def k(prev_hbm, gidx_hbm, seg_hbm, bnd_hbm, zeros_hbm, out_hbm,
          gidx_v, seg_v, loc_v, rows_v, bnd_v, acc_sh, isem, gsem, ssem):
        cid = lax.axis_index("c")
        sid = lax.axis_index("s")
        pltpu.sync_copy(bnd_hbm.at[cid], bnd_v)
        bv = bnd_v[...]

        for phase in range(NQ // NC):
            q = cid * (NQ // NC) + phase
            seg_base = q * SEG_Q
            b0 = bv[2 * phase]
            b1 = bv[2 * phase + 1]

            pltpu.sync_copy(zeros_hbm, acc_sh.at[pl.ds(sid * ROWS_PER_SUB, ROWS_PER_SUB)])
            plsc.subcore_barrier()

            nb = b1 - b0
            per = (nb + NS - 1) // NS
            s0 = b0 + sid * per
            cnt = jnp.clip(b1 - s0, 0, per)
            ng = (cnt + GI - 1) // GI

            def idx_load(p, pslot):
                off = (s0 + p * GI) * K
                pltpu.async_copy(gidx_hbm.at[pl.ds(off, GIK)], gidx_v.at[pslot],
                                 isem.at[pslot])
                pltpu.async_copy(seg_hbm.at[pl.ds(off, GIK)], seg_v.at[pslot],
                                 isem.at[pslot])

            def idx_wait(p, pslot):
                off = (s0 + p * GI) * K
                pltpu.make_async_copy(gidx_hbm.at[pl.ds(off, GIK)],
                                      gidx_v.at[pslot], isem.at[pslot]).wait()
                pltpu.make_async_copy(seg_hbm.at[pl.ds(off, GIK)],
                                      seg_v.at[pslot], isem.at[pslot]).wait()

            def gather_issue(t, j, pslot):
                rslot = j % R

                @pl.when(t >= R)
                def _():
                    pltpu.make_async_copy(rows_v.at[rslot],
                                          acc_sh.at[loc_v.at[rslot]],
                                          ssem.at[rslot]).wait()

                for jj in range(K // 16):
                    sl = pl.ds(j * K + jj * 16, 16)
                    lv = seg_v[pslot, sl] - seg_base
                    lv = jnp.where((lv < 0) | (lv >= SEG_Q), TRASH, lv)
                    loc_v[rslot, pl.ds(jj * 16, 16)] = lv
                pltpu.async_copy(
                    prev_hbm.at[gidx_v.at[pslot, pl.ds(j * K, K)]],
                    rows_v.at[rslot], gsem.at[rslot])

            def scatter_issue(j, pslot):
                rslot = j % R
                pltpu.make_async_copy(
                    prev_hbm.at[gidx_v.at[pslot, pl.ds(j * K, K)]],
                    rows_v.at[rslot], gsem.at[rslot]).wait()
                pltpu.async_copy(rows_v.at[rslot], acc_sh.at[loc_v.at[rslot]],
                                 ssem.at[rslot], add=True)

            def group_body(p, pslot):
                @pl.when(p < ng)
                def _():
                    idx_wait(p, pslot)

                for j in range(GI):
                    t = p * GI + j
                    if j == 0:
                        @pl.when((t >= 1) & (t - 1 < cnt))
                        def _():
                            scatter_issue(GI - 1, 1 - pslot)

                        @pl.when(p + 1 < ng)
                        def _():
                            idx_load(p + 1, 1 - pslot)

                        @pl.when(t < cnt)
                        def _():
                            gather_issue(t, 0, pslot)
                    else:

                        @pl.when(t < cnt)
                        def _():
                            gather_issue(t, j, pslot)

                        @pl.when(t - 1 < cnt)
                        def _():
                            scatter_issue(j - 1, pslot)

            def pair_body(i, carry):
                group_body(2 * i, 0)
                group_body(2 * i + 1, 1)
                return carry

            @pl.when(ng > 0)
            def _():
                idx_load(0, 0)

            lax.fori_loop(0, ng // 2 + 1, pair_body, 0)

            for r in range(R):
                @pl.when(r < cnt)
                def _():
                    pltpu.make_async_copy(rows_v.at[r],
                                          acc_sh.at[loc_v.at[r]],
                                          ssem.at[r]).wait()

            plsc.subcore_barrier()

            pltpu.sync_copy(
                acc_sh.at[pl.ds(sid * ROWS_PER_SUB, ROWS_PER_SUB)],
                out_hbm.at[pl.ds(seg_base + sid * ROWS_PER_SUB, ROWS_PER_SUB)],
            )
            plsc.subcore_barrier()

    return k(prev_values, gather_idx, segment_ids, bounds, zeros)


def _tc_matmul_tanh(seg_sum, W):
    BM = 2048

    def body(s_ref, w_ref, o_ref):
        o_ref[...] = jnp.tanh(
            jnp.dot(s_ref[...], w_ref[...], preferred_element_type=jnp.float32)
        )

    return pl.pallas_call(
        body,
        grid=(NUM_NEURONS // BM,),
        in_specs=[
            pl.BlockSpec((BM, D), lambda i: (i, 0)),
            pl.BlockSpec((D, D), lambda i: (0, 0)),
        ],
        out_specs=pl.BlockSpec((BM, D), lambda i: (i, 0)),
        out_shape=jax.ShapeDtypeStruct((NUM_NEURONS, D), jnp.float32),
    )(seg_sum, W)


def kernel(prev_values, W, gather_idx, segment_ids):
    gidx = gather_idx.astype(jnp.int32)
    seg = segment_ids.astype(jnp.int32)

    pad_i = jnp.zeros((PAD,), jnp.int32)
    gidx_p = jnp.concatenate([gidx, pad_i])
    seg_p = jnp.concatenate([seg, pad_i])

    edges = jnp.arange(1, NQ, dtype=jnp.int32) * SEG_Q
    cut = jnp.searchsorted(seg, edges, side="left").astype(jnp.int32)
    starts = jnp.concatenate([jnp.zeros((1,), jnp.int32), cut // K])
    ends = jnp.concatenate([(cut + K - 1) // K, jnp.full((1,), NB, jnp.int32)])
    per_q = jnp.stack([starts, ends], axis=1).reshape(NC, 2 * (NQ // NC))
    bounds = jnp.concatenate(
        [per_q, jnp.zeros((NC, 16 - 2 * (NQ // NC)), jnp.int32)], axis=1
    )
    zeros = jnp.zeros((ROWS_PER_SUB, D), jnp.float32)

    seg_sum = _sc_gather_segment_sum(prev_values, gidx_p, seg_p, bounds, zeros)
    return _tc_matmul_tanh(seg_sum, W)


# speedup vs baseline: 16.5697x; 1.0964x over previous
import functools

import jax
import jax.numpy as jnp
from jax import lax
from jax.experimental import pallas as pl
from jax.experimental.pallas import tpu as pltpu
from jax.experimental.pallas import tpu_sc as plsc

NUM_SOURCES = 100000
D = 128
NUM_NEURONS = 32768
NUM_CONN = 524288

NC = 2
NS = 16
K = 128
NB = NUM_CONN // K
NQ = 8
SEG_Q = NUM_NEURONS // NQ
ROWS_PER_SUB = SEG_Q // NS
TRASH = SEG_Q
ACC_ROWS = SEG_Q + 8
R = 5
GI = 5
GIK = GI * K
PAD = (GI - 1) * K
LAG = 2


def _sc_gather_segment_sum(prev_values, gather_idx, segment_ids, bounds, zeros):
    mesh = plsc.VectorSubcoreMesh(core_axis_name="c", subcore_axis_name="s")

    @functools.partial(
        pl.kernel,
        mesh=mesh,
        out_type=jax.ShapeDtypeStruct((NUM_NEURONS, D), jnp.float32),
        scratch_types=[
            pltpu.VMEM((2, GIK), jnp.int32),
            pltpu.VMEM((2, GIK), jnp.int32),
            pltpu.VMEM((R, K), jnp.int32),
            pltpu.VMEM((R, K, D), jnp.float32),
            pltpu.VMEM((16,), jnp.int32),
            pltpu.VMEM_SHARED((ACC_ROWS, D), jnp.float32),
            pltpu.SemaphoreType.DMA((2,)),
            pltpu.SemaphoreType.DMA((R,)),
            pltpu.SemaphoreType.DMA((R,)),
        ],
    )
    def k(prev_hbm, gidx_hbm, seg_hbm, bnd_hbm, zeros_hbm, out_hbm,
          gidx_v, seg_v, loc_v, rows_v, bnd_v, acc_sh, isem, gsem, ssem):
        cid = lax.axis_index("c")
        sid = lax.axis_index("s")
        pltpu.sync_copy(bnd_hbm.at[cid], bnd_v)
        bv = bnd_v[...]

        for phase in range(NQ // NC):
            q = cid * (NQ // NC) + phase
            seg_base = q * SEG_Q
            b0 = bv[2 * phase]
            b1 = bv[2 * phase + 1]

            pltpu.sync_copy(zeros_hbm, acc_sh.at[pl.ds(sid * ROWS_PER_SUB, ROWS_PER_SUB)])
            plsc.subcore_barrier()

            nb = b1 - b0
            per = (nb + NS - 1) // NS
            s0 = b0 + sid * per
            cnt = jnp.clip(b1 - s0, 0, per)
            ng = (cnt + GI - 1) // GI

            def idx_load(p, pslot):
                off = (s0 + p * GI) * K
                pltpu.async_copy(gidx_hbm.at[pl.ds(off, GIK)], gidx_v.at[pslot],
                                 isem.at[pslot])
                pltpu.async_copy(seg_hbm.at[pl.ds(off, GIK)], seg_v.at[pslot],
                                 isem.at[pslot])

            def idx_wait(p, pslot):
                off = (s0 + p * GI) * K
                pltpu.make_async_copy(gidx_hbm.at[pl.ds(off, GIK)],
                                      gidx_v.at[pslot], isem.at[pslot]).wait()
                pltpu.make_async_copy(seg_hbm.at[pl.ds(off, GIK)],
                                      seg_v.at[pslot], isem.at[pslot]).wait()

            def gather_issue(t, j, pslot):
                rslot = j % R

                @pl.when(t >= R)
                def _():
                    pltpu.make_async_copy(rows_v.at[rslot],
                                          acc_sh.at[loc_v.at[rslot]],
                                          ssem.at[rslot]).wait()

                for jj in range(K // 16):
                    sl = pl.ds(j * K + jj * 16, 16)
                    lv = seg_v[pslot, sl] - seg_base
                    lv = jnp.where((lv < 0) | (lv >= SEG_Q), TRASH, lv)
                    loc_v[rslot, pl.ds(jj * 16, 16)] = lv
                pltpu.async_copy(
                    prev_hbm.at[gidx_v.at[pslot, pl.ds(j * K, K)]],
                    rows_v.at[rslot], gsem.at[rslot])

            def scatter_issue(j, pslot):
                rslot = j % R
                pltpu.make_async_copy(
                    prev_hbm.at[gidx_v.at[pslot, pl.ds(j * K, K)]],
                    rows_v.at[rslot], gsem.at[rslot]).wait()
                pltpu.async_copy(rows_v.at[rslot], acc_sh.at[loc_v.at[rslot]],
                                 ssem.at[rslot], add=True)

            def group_body(p, pslot):
                @pl.when(p < ng)
                def _():
                    idx_wait(p, pslot)

                for j in range(GI):
                    t = p * GI + j

                    @pl.when(t < cnt)
                    def _():
                        gather_issue(t, j, pslot)

                    sj = j - LAG
                    sslot = pslot if sj >= 0 else 1 - pslot

                    @pl.when((t - LAG >= 0) & (t - LAG < cnt))
                    def _():
                        scatter_issue(sj % GI, sslot)

                    if j == LAG:
                        @pl.when(p + 1 < ng)
                        def _():
                            idx_load(p + 1, 1 - pslot)

            def pair_body(i, carry):
                group_body(2 * i, 0)
                group_body(2 * i + 1, 1)
                return carry

            @pl.when(ng > 0)
            def _():
                idx_load(0, 0)

            lax.fori_loop(0, ng // 2 + 1, pair_body, 0)

            for r in range(R):
                @pl.when(r < cnt)
                def _():
                    pltpu.make_async_copy(rows_v.at[r],
                                          acc_sh.at[loc_v.at[r]],
                                          ssem.at[r]).wait()

            plsc.subcore_barrier()

            pltpu.sync_copy(
                acc_sh.at[pl.ds(sid * ROWS_PER_SUB, ROWS_PER_SUB)],
                out_hbm.at[pl.ds(seg_base + sid * ROWS_PER_SUB, ROWS_PER_SUB)],
            )
            plsc.subcore_barrier()

    return k(prev_values, gather_idx, segment_ids, bounds, zeros)


def _tc_matmul_tanh(seg_sum, W):
    BM = 2048

    def body(s_ref, w_ref, o_ref):
        o_ref[...] = jnp.tanh(
            jnp.dot(s_ref[...], w_ref[...], preferred_element_type=jnp.float32)
        )

    return pl.pallas_call(
        body,
        grid=(NUM_NEURONS // BM,),
        in_specs=[
            pl.BlockSpec((BM, D), lambda i: (i, 0)),
            pl.BlockSpec((D, D), lambda i: (0, 0)),
        ],
        out_specs=pl.BlockSpec((BM, D), lambda i: (i, 0)),
        out_shape=jax.ShapeDtypeStruct((NUM_NEURONS, D), jnp.float32),
    )(seg_sum, W)


def kernel(prev_values, W, gather_idx, segment_ids):
    gidx = gather_idx.astype(jnp.int32)
    seg = segment_ids.astype(jnp.int32)

    pad_i = jnp.zeros((PAD,), jnp.int32)
    gidx_p = jnp.concatenate([gidx, pad_i])
    seg_p = jnp.concatenate([seg, pad_i])

    edges = jnp.arange(1, NQ, dtype=jnp.int32) * SEG_Q
    cut = jnp.searchsorted(seg, edges, side="left").astype(jnp.int32)
    starts = jnp.concatenate([jnp.zeros((1,), jnp.int32), cut // K])
    ends = jnp.concatenate([(cut + K - 1) // K, jnp.full((1,), NB, jnp.int32)])
    per_q = jnp.stack([starts, ends], axis=1).reshape(NC, 2 * (NQ // NC))
    bounds = jnp.concatenate(
        [per_q, jnp.zeros((NC, 16 - 2 * (NQ // NC)), jnp.int32)], axis=1
    )
    zeros = jnp.zeros((ROWS_PER_SUB, D), jnp.float32)

    seg_sum = _sc_gather_segment_sum(prev_values, gidx_p, seg_p, bounds, zeros)
    return _tc_matmul_tanh(seg_sum, W)
